# baseline (speedup 1.0000x reference)
import jax
import jax.numpy as jnp
from jax import lax
from jax.experimental import pallas as pl
from jax.experimental.pallas import tpu as pltpu
from jax.experimental.pallas import tpu_sc as plsc

S = 30.0
M = 0.4

_B = 1024
_C = 100000

_NW = 32
_ROWS_PW = _B // _NW
_CHUNK = 20000
_T = (_ROWS_PW * _C) // _CHUNK
_NBUF = 2
_VREGS = _CHUNK // 16


def _sc_body(cos_hbm, fp_hbm, out_hbm, fp_v, vals_v, *bufs_and_sems):
    bufin = bufs_and_sems[:_NBUF]
    bufout = bufs_and_sems[_NBUF:2 * _NBUF]
    in_sems = bufs_and_sems[2 * _NBUF]
    out_sems = bufs_and_sems[2 * _NBUF + 1]
    fix_sem = bufs_and_sems[2 * _NBUF + 2]

    wid = lax.axis_index("s") * 2 + lax.axis_index("c")
    base_row = wid * _ROWS_PW
    base_elem = base_row * _C

    pltpu.sync_copy(fp_hbm.at[pl.ds(base_row, _ROWS_PW)], fp_v)

    def start_in(t, b):
        pltpu.async_copy(
            cos_hbm.at[pl.ds(base_elem + t * _CHUNK, _CHUNK)],
            bufin[b],
            in_sems.at[b],
        )

    def wait_in(t, b):
        pltpu.make_async_copy(
            cos_hbm.at[pl.ds(base_elem + t * _CHUNK, _CHUNK)],
            bufin[b],
            in_sems.at[b],
        ).wait()

    def start_out(t, b):
        pltpu.async_copy(
            bufout[b],
            out_hbm.at[pl.ds(base_elem + t * _CHUNK, _CHUNK)],
            out_sems.at[b],
        )

    def wait_out(t, b):
        pltpu.make_async_copy(
            bufout[b],
            out_hbm.at[pl.ds(base_elem + t * _CHUNK, _CHUNK)],
            out_sems.at[b],
        ).wait()

    for b in range(_NBUF):
        start_in(b, b)

    def round_body(g, _):
        for b in range(_NBUF):
            t = g * _NBUF + b
            wait_in(t, b)

            @pl.when(g > 0)
            def _():
                wait_out(t - _NBUF, b)

            def vec_body(j, _):
                sl = pl.ds(j * 16, 16)
                bufout[b][sl] = bufin[b][sl] * S
                return 0

            lax.fori_loop(0, _VREGS, vec_body, 0, unroll=25)

            start_out(t, b)

            @pl.when(t + _NBUF < _T)
            def _():
                start_in(t + _NBUF, b)
        return 0

    lax.fori_loop(0, _T // _NBUF, round_body, 0)

    for b in range(_NBUF):
        wait_out(_T - _NBUF + b, b)

    pltpu.async_copy(out_hbm.at[fp_v], vals_v, fix_sem).wait()
    for k in range(_ROWS_PW // 16):
        sl = pl.ds(k * 16, 16)
        vals_v[sl] = vals_v[sl] - jnp.float32(S * M)
    pltpu.async_copy(vals_v, out_hbm.at[fp_v], fix_sem).wait()


@jax.jit
def kernel(cosine, label):
    B, C = cosine.shape
    cos_flat = cosine.reshape(B * C)
    flatpos = (jnp.arange(B, dtype=jnp.int32) * C + label.astype(jnp.int32))

    mesh = plsc.VectorSubcoreMesh(core_axis_name="c", subcore_axis_name="s")
    out_flat = pl.kernel(
        _sc_body,
        mesh=mesh,
        out_type=jax.ShapeDtypeStruct((B * C,), jnp.float32),
        scratch_types=(
            [pltpu.VMEM((_ROWS_PW,), jnp.int32), pltpu.VMEM((_ROWS_PW,), jnp.float32)]
            + [pltpu.VMEM((_CHUNK,), jnp.float32) for _ in range(2 * _NBUF)]
            + [
                pltpu.SemaphoreType.DMA((_NBUF,)),
                pltpu.SemaphoreType.DMA((_NBUF,)),
                pltpu.SemaphoreType.DMA,
            ]
        ),
    )(cos_flat, flatpos)
    return out_flat.reshape(B, C)


# speedup vs baseline: 2.7522x; 2.7522x over previous
import jax
import jax.numpy as jnp
from jax import lax
from jax.experimental import pallas as pl
from jax.experimental.pallas import tpu as pltpu

S = 30.0
M = 0.4

_ROWS = 8
_NBUF = 4


def _tc_body(cos_hbm, lab_ref, out_hbm, *bufs_and_sems):
    bufin = bufs_and_sems[:_NBUF]
    bufout = bufs_and_sems[_NBUF:2 * _NBUF]
    in_sems = bufs_and_sems[2 * _NBUF]
    out_sems = bufs_and_sems[2 * _NBUF + 1]

    B, C = cos_hbm.shape
    nchunks = B // _ROWS

    def start_in(t, b):
        pltpu.make_async_copy(
            cos_hbm.at[pl.ds(t * _ROWS, _ROWS)], bufin[b], in_sems.at[b]
        ).start()

    def wait_in(t, b):
        pltpu.make_async_copy(
            cos_hbm.at[pl.ds(t * _ROWS, _ROWS)], bufin[b], in_sems.at[b]
        ).wait()

    def start_out(t, b):
        pltpu.make_async_copy(
            bufout[b], out_hbm.at[pl.ds(t * _ROWS, _ROWS)], out_sems.at[b]
        ).start()

    def wait_out(t, b):
        pltpu.make_async_copy(
            bufout[b], out_hbm.at[pl.ds(t * _ROWS, _ROWS)], out_sems.at[b]
        ).wait()

    for b in range(_NBUF):
        start_in(b, b)

    cols = lax.broadcasted_iota(jnp.int32, (_ROWS, C), 1)

    def round_body(g, _):
        for b in range(_NBUF):
            t = g * _NBUF + b
            wait_in(t, b)

            @pl.when(g > 0)
            def _():
                wait_out(t - _NBUF, b)

            labs = lab_ref[pl.ds(t * _ROWS, _ROWS), :]
            mask = cols == labs
            bufout[b][...] = bufin[b][...] * S - jnp.where(
                mask, jnp.float32(S * M), jnp.float32(0.0)
            )

            start_out(t, b)

            @pl.when(t + _NBUF < nchunks)
            def _():
                start_in(t + _NBUF, b)
        return 0

    lax.fori_loop(0, nchunks // _NBUF, round_body, 0)

    for b in range(_NBUF):
        wait_out(nchunks - _NBUF + b, b)


@jax.jit
def kernel(cosine, label):
    B, C = cosine.shape
    lab2d = label.astype(jnp.int32).reshape(B, 1)

    return pl.pallas_call(
        _tc_body,
        in_specs=[
            pl.BlockSpec(memory_space=pltpu.HBM),
            pl.BlockSpec(memory_space=pltpu.VMEM),
        ],
        out_specs=pl.BlockSpec(memory_space=pltpu.HBM),
        out_shape=jax.ShapeDtypeStruct((B, C), jnp.float32),
        scratch_shapes=(
            [pltpu.VMEM((_ROWS, C), jnp.float32) for _ in range(2 * _NBUF)]
            + [pltpu.SemaphoreType.DMA((_NBUF,)), pltpu.SemaphoreType.DMA((_NBUF,))]
        ),
    )(cosine, lab2d)


# speedup vs baseline: 2.7527x; 1.0002x over previous
import jax
import jax.numpy as jnp
from jax import lax
from jax.experimental import pallas as pl
from jax.experimental.pallas import tpu as pltpu

S = 30.0
M = 0.4

_ROWS = 4
_NBUF = 16


def _tc_body(cos_hbm, lab_ref, out_hbm, *bufs_and_sems):
    bufin = bufs_and_sems[:_NBUF]
    bufout = bufs_and_sems[_NBUF:2 * _NBUF]
    in_sems = bufs_and_sems[2 * _NBUF]
    out_sems = bufs_and_sems[2 * _NBUF + 1]

    B, C = cos_hbm.shape
    nchunks = B // _ROWS

    def start_in(t, b):
        pltpu.make_async_copy(
            cos_hbm.at[pl.ds(t * _ROWS, _ROWS)], bufin[b], in_sems.at[b]
        ).start()

    def wait_in(t, b):
        pltpu.make_async_copy(
            cos_hbm.at[pl.ds(t * _ROWS, _ROWS)], bufin[b], in_sems.at[b]
        ).wait()

    def start_out(t, b):
        pltpu.make_async_copy(
            bufout[b], out_hbm.at[pl.ds(t * _ROWS, _ROWS)], out_sems.at[b]
        ).start()

    def wait_out(t, b):
        pltpu.make_async_copy(
            bufout[b], out_hbm.at[pl.ds(t * _ROWS, _ROWS)], out_sems.at[b]
        ).wait()

    for b in range(_NBUF):
        start_in(b, b)

    cols = lax.broadcasted_iota(jnp.int32, (_ROWS, C), 1)

    def round_body(g, _):
        for b in range(_NBUF):
            t = g * _NBUF + b
            wait_in(t, b)

            @pl.when(g > 0)
            def _():
                wait_out(t - _NBUF, b)

            labs = lab_ref[pl.ds(t * _ROWS, _ROWS), :]
            mask = cols == labs
            bufout[b][...] = bufin[b][...] * S - jnp.where(
                mask, jnp.float32(S * M), jnp.float32(0.0)
            )

            start_out(t, b)

            @pl.when(t + _NBUF < nchunks)
            def _():
                start_in(t + _NBUF, b)
        return 0

    lax.fori_loop(0, nchunks // _NBUF, round_body, 0)

    for b in range(_NBUF):
        wait_out(nchunks - _NBUF + b, b)


@jax.jit
def kernel(cosine, label):
    B, C = cosine.shape
    lab2d = label.astype(jnp.int32).reshape(B, 1)

    return pl.pallas_call(
        _tc_body,
        in_specs=[
            pl.BlockSpec(memory_space=pltpu.HBM),
            pl.BlockSpec(memory_space=pltpu.VMEM),
        ],
        out_specs=pl.BlockSpec(memory_space=pltpu.HBM),
        out_shape=jax.ShapeDtypeStruct((B, C), jnp.float32),
        scratch_shapes=(
            [pltpu.VMEM((_ROWS, C), jnp.float32) for _ in range(2 * _NBUF)]
            + [pltpu.SemaphoreType.DMA((_NBUF,)), pltpu.SemaphoreType.DMA((_NBUF,))]
        ),
    )(cosine, lab2d)


# speedup vs baseline: 2.7648x; 1.0044x over previous
import jax
import jax.numpy as jnp
from jax.experimental import pallas as pl

S = 30.0
M = 0.4

_BLOCK_B = 256
_BLOCK_C = 8192


def _mcp_block(cosine_ref, label_ref, out_ref):
    j = pl.program_id(1)
    cols = jax.lax.broadcasted_iota(jnp.int32, cosine_ref.shape, 1) + j * _BLOCK_C
    mask = cols == label_ref[...]
    out_ref[...] = cosine_ref[...] * S - jnp.where(mask, S * M, jnp.float32(0.0))


@jax.jit
def kernel(cosine, label):
    B, C = cosine.shape
    label2d = label.astype(jnp.int32).reshape(B, 1)
    nb = pl.cdiv(B, _BLOCK_B)
    nc = pl.cdiv(C, _BLOCK_C)
    return pl.pallas_call(
        _mcp_block,
        grid=(nb, nc),
        in_specs=[
            pl.BlockSpec((_BLOCK_B, _BLOCK_C), lambda i, j: (i, j)),
            pl.BlockSpec((_BLOCK_B, 1), lambda i, j: (i, 0)),
        ],
        out_specs=pl.BlockSpec((_BLOCK_B, _BLOCK_C), lambda i, j: (i, j)),
        out_shape=jax.ShapeDtypeStruct((B, C), cosine.dtype),
    )(cosine, label2d)
